# Initial kernel scaffold; baseline (speedup 1.0000x reference)
import jax
import jax.numpy as jnp
from jax.experimental import pallas as pl


def kernel(sensor_tokens, timestamps, channel_embed, pos_embed, month_table):
    raise NotImplementedError("write your pallas kernel here")


# speedup vs baseline: 2.3760x; 2.3760x over previous
import jax
import jax.numpy as jnp
from jax.experimental import pallas as pl
from jax.experimental.pallas import tpu as pltpu


def _embed_add_kernel(months_ref, x_ref, ce_ref, pe_ref, mt_ref, o_ref):
    bi = pl.program_id(0)
    ti = pl.program_id(1)
    n = ce_ref.shape[-1]
    m = months_ref[bi, ti]
    x = x_ref[...]
    ce = ce_ref[...]
    pe = pe_ref[ti, :]
    me = mt_ref[m, :]
    o_ref[..., 0:n] = x[..., 0:n] + ce[None, None, None, :, :]
    o_ref[..., n:2 * n] = x[..., n:2 * n] + pe[None, None, None, None, :]
    o_ref[..., 2 * n:3 * n] = x[..., 2 * n:3 * n] + me[None, None, None, None, :]
    o_ref[..., 3 * n:] = x[..., 3 * n:]


def kernel(sensor_tokens, timestamps, channel_embed, pos_embed, month_table):
    b, h, w, t, b_s, d = sensor_tokens.shape
    n = d // 4
    hw = h * w
    x = sensor_tokens.reshape(b, hw, t, b_s, d)
    months = timestamps[:, :, 1].astype(jnp.int32)

    grid = (b, t)
    out = pl.pallas_call(
        _embed_add_kernel,
        grid_spec=pltpu.PrefetchScalarGridSpec(
            num_scalar_prefetch=1,
            grid=grid,
            in_specs=[
                pl.BlockSpec((1, hw, 1, b_s, d), lambda i, j, m_ref: (i, 0, j, 0, 0)),
                pl.BlockSpec((b_s, n), lambda i, j, m_ref: (0, 0)),
                pl.BlockSpec((t, n), lambda i, j, m_ref: (0, 0)),
                pl.BlockSpec(month_table.shape, lambda i, j, m_ref: (0, 0)),
            ],
            out_specs=pl.BlockSpec((1, hw, 1, b_s, d), lambda i, j, m_ref: (i, 0, j, 0, 0)),
        ),
        out_shape=jax.ShapeDtypeStruct(x.shape, x.dtype),
        compiler_params=pltpu.CompilerParams(
            dimension_semantics=("arbitrary", "arbitrary"),
        ),
    )(months, x, channel_embed, pos_embed[:t], month_table)
    return out.reshape(b, h, w, t, b_s, d)


# speedup vs baseline: 2.3903x; 1.0060x over previous
import jax
import jax.numpy as jnp
from jax.experimental import pallas as pl
from jax.experimental.pallas import tpu as pltpu


def _embed_add_kernel(months_ref, x_ref, ce_ref, pe_ref, mt_ref, o_ref):
    bi = pl.program_id(0)
    t = pe_ref.shape[0]
    n = ce_ref.shape[-1]
    x = x_ref[...]
    ce = ce_ref[...]
    pe = pe_ref[...]
    me = jnp.stack([mt_ref[months_ref[bi, tt], :] for tt in range(t)])
    o_ref[..., 0:n] = x[..., 0:n] + ce[None, None, None, :, :]
    o_ref[..., n:2 * n] = x[..., n:2 * n] + pe[None, None, :, None, :]
    o_ref[..., 2 * n:3 * n] = x[..., 2 * n:3 * n] + me[None, None, :, None, :]
    o_ref[..., 3 * n:] = x[..., 3 * n:]


def kernel(sensor_tokens, timestamps, channel_embed, pos_embed, month_table):
    b, h, w, t, b_s, d = sensor_tokens.shape
    n = d // 4
    hw = h * w
    br = 32
    x = sensor_tokens.reshape(b, hw, t, b_s, d)
    months = timestamps[:, :, 1].astype(jnp.int32)

    grid = (b, hw // br)
    out = pl.pallas_call(
        _embed_add_kernel,
        grid_spec=pltpu.PrefetchScalarGridSpec(
            num_scalar_prefetch=1,
            grid=grid,
            in_specs=[
                pl.BlockSpec((1, br, t, b_s, d), lambda i, j, m_ref: (i, j, 0, 0, 0)),
                pl.BlockSpec((b_s, n), lambda i, j, m_ref: (0, 0)),
                pl.BlockSpec((t, n), lambda i, j, m_ref: (0, 0)),
                pl.BlockSpec(month_table.shape, lambda i, j, m_ref: (0, 0)),
            ],
            out_specs=pl.BlockSpec((1, br, t, b_s, d), lambda i, j, m_ref: (i, j, 0, 0, 0)),
        ),
        out_shape=jax.ShapeDtypeStruct(x.shape, x.dtype),
        compiler_params=pltpu.CompilerParams(
            dimension_semantics=("arbitrary", "arbitrary"),
        ),
    )(months, x, channel_embed, pos_embed[:t], month_table)
    return out.reshape(b, h, w, t, b_s, d)
